# Initial kernel scaffold; baseline (speedup 1.0000x reference)
import jax
import jax.numpy as jnp
from jax.experimental import pallas as pl


def kernel(x, w1, b1, w2, b2):
    raise NotImplementedError("write your pallas kernel here")


# speedup vs baseline: 1.0152x; 1.0152x over previous
import functools

import jax
import jax.numpy as jnp
from jax.experimental import pallas as pl
from jax.experimental.pallas import tpu as pltpu


def _se_gate_kernel(x_ref, w1_ref, b1_ref, w2_ref, b2_ref, o_ref, *, inv_hw):
    pooled = jnp.sum(x_ref[...].astype(jnp.float32), axis=-1) * inv_hw
    h = jnp.dot(pooled, w1_ref[...].astype(jnp.float32),
                preferred_element_type=jnp.float32) + b1_ref[...]
    h = jnp.maximum(h, 0.0)
    g = jnp.dot(h, w2_ref[...].astype(jnp.float32),
                preferred_element_type=jnp.float32) + b2_ref[...]
    o_ref[...] = jnp.broadcast_to(g[:, :, None], o_ref.shape).astype(o_ref.dtype)


def kernel(x, w1, b1, w2, b2):
    B, C, H, W = x.shape
    HW = H * W
    Ch = w1.shape[1]
    itemsize = jnp.dtype(x.dtype).itemsize

    x3 = x.reshape(B, C, HW)
    b1_2d = b1.reshape(1, Ch).astype(jnp.float32)
    b2_2d = b2.reshape(1, C).astype(jnp.float32)

    slab_budget = 14 << 20
    tb = B
    for cand in (8, 4, 2, 1):
        if B % cand == 0 and cand * C * HW * itemsize <= slab_budget:
            tb = cand
            break
    num_b = B // tb

    out = pl.pallas_call(
        functools.partial(_se_gate_kernel, inv_hw=1.0 / HW),
        out_shape=jax.ShapeDtypeStruct((B, C, HW), x.dtype),
        grid=(num_b,),
        in_specs=[
            pl.BlockSpec((tb, C, HW), lambda i: (i, 0, 0)),
            pl.BlockSpec((C, Ch), lambda i: (0, 0)),
            pl.BlockSpec((1, Ch), lambda i: (0, 0)),
            pl.BlockSpec((Ch, C), lambda i: (0, 0)),
            pl.BlockSpec((1, C), lambda i: (0, 0)),
        ],
        out_specs=pl.BlockSpec((tb, C, HW), lambda i: (i, 0, 0)),
        compiler_params=pltpu.CompilerParams(
            dimension_semantics=("parallel",),
            vmem_limit_bytes=64 << 20),
        cost_estimate=pl.CostEstimate(
            flops=B * C * HW + 4 * B * C * Ch,
            transcendentals=0,
            bytes_accessed=2 * B * C * HW * itemsize),
    )(x3, w1, b1_2d, w2, b2_2d)

    return out.reshape(B, C, H, W)


# speedup vs baseline: 3.0182x; 2.9729x over previous
import functools

import jax
import jax.numpy as jnp
from jax.experimental import pallas as pl
from jax.experimental.pallas import tpu as pltpu


def _se_gate_kernel(x_ref, w1t_ref, b1_ref, w2_ref, b2_ref, o_ref, acc_ref,
                    *, num_k, inv_hw):
    j = pl.program_id(1)

    @pl.when(j == 0)
    def _init():
        acc_ref[...] = jnp.zeros_like(acc_ref)

    @pl.when(j < num_k)
    def _accumulate():
        acc_ref[...] += jnp.sum(x_ref[...].astype(jnp.float32), axis=0)

    @pl.when(j == num_k - 1)
    def _mlp():
        pooled = acc_ref[...] * inv_hw
        h = jax.lax.dot_general(pooled, w1t_ref[...].astype(jnp.float32),
                                (((1,), (1,)), ((), ())),
                                preferred_element_type=jnp.float32) + b1_ref[...]
        h = jnp.maximum(h, 0.0)
        g = jnp.dot(h, w2_ref[...].astype(jnp.float32),
                    preferred_element_type=jnp.float32) + b2_ref[...]
        acc_ref[...] = g

    @pl.when(j >= num_k)
    def _store():
        o_ref[...] = jnp.broadcast_to(
            acc_ref[...][None], o_ref.shape).astype(o_ref.dtype)


def _pick_tile(n, target):
    best = 1
    for d in range(1, n + 1):
        if n % d == 0 and d <= target:
            best = d
    return best if best > 0 else n


def kernel(x, w1, b1, w2, b2):
    B, C, H, W = x.shape
    HW = H * W
    Ch = w1.shape[1]
    itemsize = jnp.dtype(x.dtype).itemsize

    xt = jnp.transpose(x, (2, 3, 0, 1)).reshape(HW, B, C)
    w1t = jnp.transpose(w1)
    b1_2d = b1.reshape(1, Ch).astype(jnp.float32)
    b2_2d = b2.reshape(1, C).astype(jnp.float32)

    tb = _pick_tile(B, 8)
    thw = _pick_tile(HW, max(1, (3 << 20) // max(1, tb * C * itemsize)))
    num_b = B // tb
    num_k = HW // thw

    out = pl.pallas_call(
        functools.partial(_se_gate_kernel, num_k=num_k, inv_hw=1.0 / HW),
        out_shape=jax.ShapeDtypeStruct((HW, B, C), x.dtype),
        grid=(num_b, 2 * num_k),
        in_specs=[
            pl.BlockSpec((thw, tb, C), lambda i, j: (jnp.minimum(j, num_k - 1), i, 0)),
            pl.BlockSpec((Ch, C), lambda i, j: (0, 0)),
            pl.BlockSpec((1, Ch), lambda i, j: (0, 0)),
            pl.BlockSpec((Ch, C), lambda i, j: (0, 0)),
            pl.BlockSpec((1, C), lambda i, j: (0, 0)),
        ],
        out_specs=pl.BlockSpec(
            (thw, tb, C), lambda i, j: (jnp.maximum(j - num_k, 0), i, 0)),
        scratch_shapes=[pltpu.VMEM((tb, C), jnp.float32)],
        compiler_params=pltpu.CompilerParams(
            dimension_semantics=("parallel", "arbitrary"),
            vmem_limit_bytes=64 << 20),
        cost_estimate=pl.CostEstimate(
            flops=B * C * HW + 4 * B * C * Ch,
            transcendentals=0,
            bytes_accessed=2 * B * C * HW * itemsize),
    )(xt, w1t, b1_2d, w2, b2_2d)

    return jnp.transpose(out.reshape(H, W, B, C), (2, 3, 0, 1))


# speedup vs baseline: 4.5602x; 1.5109x over previous
import functools

import jax
import jax.numpy as jnp
from jax.experimental import pallas as pl
from jax.experimental.pallas import tpu as pltpu


def _se_gate_kernel(x_ref, w1t_ref, b1_ref, w2_ref, b2_ref, o_ref, acc_ref,
                    *, inv_hw):
    i = pl.program_id(0)
    nb = pl.num_programs(0) - 1
    cur = jax.lax.rem(i, 2)
    prev = jax.lax.rem(i + 1, 2)

    @pl.when(i < nb)
    def _pool_mlp():
        pooled = jnp.sum(x_ref[...].astype(jnp.float32), axis=0) * inv_hw
        h = jax.lax.dot_general(pooled, w1t_ref[...].astype(jnp.float32),
                                (((1,), (1,)), ((), ())),
                                preferred_element_type=jnp.float32) + b1_ref[...]
        h = jnp.maximum(h, 0.0)
        g = jnp.dot(h, w2_ref[...].astype(jnp.float32),
                    preferred_element_type=jnp.float32) + b2_ref[...]
        acc_ref[pl.ds(cur, 1)] = g[None]

    @pl.when(i >= 1)
    def _store():
        o_ref[...] = jnp.broadcast_to(
            acc_ref[pl.ds(prev, 1)], o_ref.shape).astype(o_ref.dtype)


def _pick_tile(n, target):
    best = 1
    for d in range(1, n + 1):
        if n % d == 0 and d <= target:
            best = d
    return best


def kernel(x, w1, b1, w2, b2):
    B, C, H, W = x.shape
    HW = H * W
    Ch = w1.shape[1]
    itemsize = jnp.dtype(x.dtype).itemsize

    xt = jnp.transpose(x, (2, 3, 0, 1)).reshape(HW, B, C)
    w1t = jnp.transpose(w1)
    b1_2d = b1.reshape(1, Ch).astype(jnp.float32)
    b2_2d = b2.reshape(1, C).astype(jnp.float32)

    tb = _pick_tile(B, max(1, (56 << 20) // max(1, 4 * HW * C * itemsize)))
    nb = B // tb

    out = pl.pallas_call(
        functools.partial(_se_gate_kernel, inv_hw=1.0 / HW),
        out_shape=jax.ShapeDtypeStruct((HW, B, C), x.dtype),
        grid=(nb + 1,),
        in_specs=[
            pl.BlockSpec((HW, tb, C), lambda i: (0, jnp.minimum(i, nb - 1), 0)),
            pl.BlockSpec((Ch, C), lambda i: (0, 0)),
            pl.BlockSpec((1, Ch), lambda i: (0, 0)),
            pl.BlockSpec((Ch, C), lambda i: (0, 0)),
            pl.BlockSpec((1, C), lambda i: (0, 0)),
        ],
        out_specs=pl.BlockSpec(
            (HW, tb, C), lambda i: (0, jnp.maximum(i - 1, 0), 0)),
        scratch_shapes=[pltpu.VMEM((2, tb, C), jnp.float32)],
        compiler_params=pltpu.CompilerParams(
            dimension_semantics=("arbitrary",),
            vmem_limit_bytes=64 << 20),
        cost_estimate=pl.CostEstimate(
            flops=B * C * HW + 4 * B * C * Ch,
            transcendentals=0,
            bytes_accessed=2 * B * C * HW * itemsize),
    )(xt, w1t, b1_2d, w2, b2_2d)

    return jnp.transpose(out.reshape(H, W, B, C), (2, 3, 0, 1))
